# baseline (speedup 1.0000x reference)
import functools

import jax
import jax.numpy as jnp
from jax import lax
from jax.experimental import pallas as pl
from jax.experimental.pallas import tpu as pltpu
from jax.experimental.pallas import tpu_sc as plsc

BATCH = 16384
D = 3200
NUM_GROUPS = 10
SLICES_PER_GROUP = 5
SLICE_W = 32
GROUP_W = SLICES_PER_GROUP * SLICE_W
USED_COLS = NUM_GROUPS * GROUP_W

_NUM_TC_GROUPS = 9


_TC_BLK = 1024
_NREFS = 13


def _tc_body(*refs):
    xs = refs[:_NREFS]
    out_refs = refs[_NREFS:]
    for i in range(_NUM_TC_GROUPS):
        g = i
        parts = []
        for j in range(SLICES_PER_GROUP):
            chunk = j * NUM_GROUPS + g
            r, o = divmod(chunk, 4)
            parts.append(xs[r][:, o * SLICE_W:(o + 1) * SLICE_W])
        out_refs[i][...] = jnp.concatenate(parts, axis=1)


_tc_call = pl.pallas_call(
    _tc_body,
    grid=(BATCH // _TC_BLK,),
    in_specs=[
        pl.BlockSpec((_TC_BLK, 128), lambda i, c=c: (i, c))
        for c in range(_NREFS)
    ],
    out_specs=[
        pl.BlockSpec((_TC_BLK, GROUP_W), lambda i: (i, 0))
        for _ in range(_NUM_TC_GROUPS)
    ],
    out_shape=[
        jax.ShapeDtypeStruct((BATCH, GROUP_W), jnp.float32)
        for _ in range(_NUM_TC_GROUPS)
    ],
)


def _tc_args(input_tensor):
    return [input_tensor] * _NREFS


_SC_GROUPS = tuple(range(_NUM_TC_GROUPS, NUM_GROUPS))
_NSC = len(_SC_GROUPS)

_INFO = plsc.get_sparse_core_info()
_NUM_WORKERS = _INFO.num_cores * _INFO.num_subcores
_ROWS_PER_WORKER = BATCH // _NUM_WORKERS
_SC_R = 128
_SC_NCHUNK = _ROWS_PER_WORKER // _SC_R

_MESH = plsc.VectorSubcoreMesh(core_axis_name="c", subcore_axis_name="s")


def _make_sc_call():
    @functools.partial(
        pl.kernel,
        mesh=_MESH,
        out_type=tuple(
            jax.ShapeDtypeStruct((BATCH, GROUP_W), jnp.float32)
            for _ in range(_NSC)
        ),
        scratch_types=[
            pltpu.VMEM((2, _NSC, _SC_R, GROUP_W), jnp.float32),
            pltpu.SemaphoreType.DMA,
            pltpu.SemaphoreType.DMA,
            pltpu.SemaphoreType.DMA,
            pltpu.SemaphoreType.DMA,
        ],
        compiler_params=pltpu.CompilerParams(use_tc_tiling_on_sc=False),
    )
    def _sc_slice_cat(in_hbm, *rest):
        out_hbms = rest[:_NSC]
        buf = rest[_NSC]
        sems_in = rest[_NSC + 1:_NSC + 3]
        sems_out = rest[_NSC + 3:_NSC + 5]
        wid = lax.axis_index("s") * _INFO.num_cores + lax.axis_index("c")
        base = wid * _ROWS_PER_WORKER

        def in_copies(k, b):
            row0 = base + k * _SC_R
            cps = []
            for i, g in enumerate(_SC_GROUPS):
                for j in range(SLICES_PER_GROUP):
                    src_col = (j * NUM_GROUPS + g) * SLICE_W
                    cps.append(
                        pltpu.make_async_copy(
                            in_hbm.at[pl.ds(row0, _SC_R), pl.ds(src_col, SLICE_W)],
                            buf.at[b, i, :, pl.ds(j * SLICE_W, SLICE_W)],
                            sems_in[b],
                        )
                    )
            return cps

        def out_copies(k, b):
            row0 = base + k * _SC_R
            return [
                pltpu.make_async_copy(
                    buf.at[b, i],
                    out_hbms[i].at[pl.ds(row0, _SC_R), :],
                    sems_out[b],
                )
                for i in range(_NSC)
            ]

        def step(k, b):
            @pl.when(k + 1 < _SC_NCHUNK)
            def _():
                for c in in_copies(k + 1, 1 - b):
                    c.start()

            for c in in_copies(k, b):
                c.wait()

            @pl.when(k >= 2)
            def _():
                for c in out_copies(k - 2, b):
                    c.wait()

            for c in out_copies(k, b):
                c.start()

        for c in in_copies(0, 0):
            c.start()
        for k0 in range(_SC_NCHUNK // 2):
            step(2 * k0, 0)
            step(2 * k0 + 1, 1)
        for b in (0, 1):
            for c in out_copies(_SC_NCHUNK - 2 + b, b):
                c.wait()

    return _sc_slice_cat


_sc_call = _make_sc_call() if _NSC else None


def kernel(input_tensor):
    if _NSC:
        sc_outs = _sc_call(input_tensor)
        tc_outs = _tc_call(*_tc_args(input_tensor))
        return (*tc_outs, *sc_outs)
    tc_outs = _tc_call(*_tc_args(input_tensor))
    return tuple(tc_outs)


# speedup vs baseline: 1.9341x; 1.9341x over previous
import jax
import jax.numpy as jnp
from jax.experimental import pallas as pl

BATCH = 16384
D = 3200
NUM_GROUPS = 10
SLICES_PER_GROUP = 5
SLICE_W = 32
GROUP_W = SLICES_PER_GROUP * SLICE_W

_TC_BLK = 1024
_NREFS = 13
_VIEW_W = 128
_CHUNKS_PER_VIEW = _VIEW_W // SLICE_W


def _tc_body(*refs):
    xs = refs[:_NREFS]
    out_refs = refs[_NREFS:]
    for g in range(NUM_GROUPS):
        parts = []
        for j in range(SLICES_PER_GROUP):
            chunk = j * NUM_GROUPS + g
            r, o = divmod(chunk, _CHUNKS_PER_VIEW)
            parts.append(xs[r][:, o * SLICE_W:(o + 1) * SLICE_W])
        out_refs[g][...] = jnp.concatenate(parts, axis=1)


_tc_call = pl.pallas_call(
    _tc_body,
    grid=(BATCH // _TC_BLK,),
    in_specs=[
        pl.BlockSpec((_TC_BLK, _VIEW_W), lambda i, c=c: (i, c))
        for c in range(_NREFS)
    ],
    out_specs=[
        pl.BlockSpec((_TC_BLK, GROUP_W), lambda i: (i, 0))
        for _ in range(NUM_GROUPS)
    ],
    out_shape=[
        jax.ShapeDtypeStruct((BATCH, GROUP_W), jnp.float32)
        for _ in range(NUM_GROUPS)
    ],
)


def kernel(input_tensor):
    return tuple(_tc_call(*([input_tensor] * _NREFS)))
